# baseline (speedup 1.0000x reference)
import functools

import jax
import jax.numpy as jnp
from jax import lax
from jax.experimental import pallas as pl
from jax.experimental.pallas import tpu as pltpu
from jax.experimental.pallas import tpu_sc as plsc

_E = 8
_NEG = -1e30
_BM = 256
_NW = 32
_L = 16
_CPW = 128


def _gate_body(gw_ref, x_ref, s0_ref, s1_ref, p0_ref, p1_ref, sp_ref,
               xc_ref):
    T = x_ref.shape[0]
    lg = lax.dot_general(x_ref[...], gw_ref[...], (((1,), (1,)), ((), ())),
                         preferred_element_type=jnp.float32)
    lane = lax.broadcasted_iota(jnp.int32, lg.shape, 1)
    lg = jnp.where(lane < _E, lg, _NEG)
    m0 = jnp.max(lg, axis=1, keepdims=True)
    a0 = jnp.min(jnp.where(lg == m0, lane, 128), axis=1, keepdims=True)
    lg1 = jnp.where(lane == a0, _NEG, lg)
    m1 = jnp.max(lg1, axis=1, keepdims=True)
    a1 = jnp.min(jnp.where(lg1 == m1, lane, 128), axis=1, keepdims=True)
    p0 = 1.0 / (1.0 + jnp.exp(m1 - m0))
    p0_ref[...] = p0
    p1_ref[...] = 1.0 - p0
    xc_ref[...] = x_ref[...]

    oh0 = (lane == a0).astype(jnp.bfloat16)
    oh1 = (lane == a1).astype(jnp.bfloat16)
    tri = (lax.broadcasted_iota(jnp.int32, (T, T), 0) >
           lax.broadcasted_iota(jnp.int32, (T, T), 1)
           ).astype(jnp.bfloat16)
    cum0 = lax.dot_general(tri, oh0, (((1,), (0,)), ((), ())),
                           preferred_element_type=jnp.float32)
    cum1 = lax.dot_general(tri, oh1, (((1,), (0,)), ((), ())),
                           preferred_element_type=jnp.float32)
    tot0 = jnp.sum(oh0.astype(jnp.float32), axis=0, keepdims=True)
    tot1 = jnp.sum(oh1.astype(jnp.float32), axis=0, keepdims=True)

    counts = tot0 + tot1
    nbl = jnp.floor((counts + (_BM - 1)) * (1.0 / _BM))
    tril = (lax.broadcasted_iota(jnp.int32, (128, 128), 0) <
            lax.broadcasted_iota(jnp.int32, (128, 128), 1)
            ).astype(jnp.float32)
    off_blk = lax.dot_general(nbl, tril, (((1,), (0,)), ((), ())),
                              preferred_element_type=jnp.float32)
    totblk = jnp.sum(jnp.where(lane[0:1, :] < _E, nbl, 0.0), axis=1,
                     keepdims=True)
    off_rows = off_blk * _BM

    s0 = jnp.sum(jnp.where(lane == a0, off_rows + cum0, 0.0), axis=1,
                 keepdims=True)
    s1 = jnp.sum(jnp.where(lane == a1, off_rows + cum1 + tot0, 0.0), axis=1,
                 keepdims=True)
    s0_ref[...] = s0.astype(jnp.int32)
    s1_ref[...] = s1.astype(jnp.int32)

    bi = lax.broadcasted_iota(jnp.int32, (128, 128), 0).astype(jnp.float32)
    bic = jnp.minimum(bi, totblk - 1.0)
    ec = lax.broadcasted_iota(jnp.int32, (128, 128), 1)
    obc = jnp.broadcast_to(off_blk, (128, 128))
    bem = ((bic >= obc) & (ec >= 1) & (ec < _E)).astype(jnp.float32)
    be = jnp.sum(bem, axis=1, keepdims=True)
    bsub = lax.broadcasted_iota(jnp.int32, (128, 1), 0)
    sp_ref[...] = jnp.where(bsub == 24, totblk, be).astype(jnp.int32)


def _gate(xf, gwp):
    T, D = xf.shape
    return pl.pallas_call(
        _gate_body,
        in_specs=[
            pl.BlockSpec((128, D), lambda: (0, 0)),
            pl.BlockSpec((T, D), lambda: (0, 0)),
        ],
        out_specs=[
            pl.BlockSpec((T, 1), lambda: (0, 0)),
            pl.BlockSpec((T, 1), lambda: (0, 0)),
            pl.BlockSpec((T, 1), lambda: (0, 0)),
            pl.BlockSpec((T, 1), lambda: (0, 0)),
            pl.BlockSpec((128, 1), lambda: (0, 0)),
            pl.BlockSpec((T, D), lambda: (0, 0)),
        ],
        out_shape=[
            jax.ShapeDtypeStruct((T, 1), jnp.int32),
            jax.ShapeDtypeStruct((T, 1), jnp.int32),
            jax.ShapeDtypeStruct((T, 1), jnp.float32),
            jax.ShapeDtypeStruct((T, 1), jnp.float32),
            jax.ShapeDtypeStruct((128, 1), jnp.int32),
            jax.ShapeDtypeStruct((T, D), jnp.float32),
        ],
    )(gwp, xf)


def _sc_route_call(s_pair, w_pair, xc, NS):
    P2 = s_pair.shape[0]
    T, D = xc.shape
    zlen = NS // 16
    hch = _CPW // 2
    mesh = plsc.VectorSubcoreMesh(core_axis_name="c", subcore_axis_name="s")

    @functools.partial(
        pl.kernel, mesh=mesh,
        out_type=[
            jax.ShapeDtypeStruct((NS, D), jnp.float32),
            jax.ShapeDtypeStruct((2 * NS,), jnp.float32),
        ],
        scratch_types=[
            pltpu.VMEM((_CPW,), jnp.int32),
            pltpu.VMEM((2, hch), jnp.int32),
            pltpu.VMEM((_CPW,), jnp.float32),
            pltpu.VMEM((hch, D), jnp.float32),
            pltpu.VMEM((hch, D), jnp.float32),
            pltpu.VMEM((NS // 16,), jnp.float32),
            pltpu.VMEM_SHARED((NS,), jnp.float32),
            pltpu.SemaphoreType.DMA,
            pltpu.SemaphoreType.DMA,
        ],
    )
    def route_k(s_hbm, w_hbm, x_hbm, xs_hbm, wp_hbm,
                slot_v, slot2_v, w_v, rowa_v, rowb_v, zf_v, w_sp,
                sa_sem, sb_sem):
        c = lax.axis_index("c")
        s = lax.axis_index("s")
        chunk = c * 16 + s
        base_p = chunk * _CPW
        tok0 = s * _CPW

        pltpu.sync_copy(s_hbm.at[pl.ds(base_p, _CPW)], slot_v)
        pltpu.sync_copy(s_hbm.at[pl.ds(base_p, hch)], slot2_v.at[0])
        pltpu.sync_copy(s_hbm.at[pl.ds(base_p + hch, hch)], slot2_v.at[1])
        pltpu.sync_copy(w_hbm.at[pl.ds(base_p, _CPW)], w_v)

        pltpu.sync_copy(x_hbm.at[pl.ds(tok0, hch)], rowa_v)
        cpa = pltpu.async_copy(rowa_v, xs_hbm.at[slot2_v.at[0]], sa_sem)
        pltpu.sync_copy(x_hbm.at[pl.ds(tok0 + hch, hch)], rowb_v)
        cpb = pltpu.async_copy(rowb_v, xs_hbm.at[slot2_v.at[1]], sb_sem)
        cpa.wait()
        cpb.wait()

        zf = jnp.zeros((_L,), jnp.float32)
        for j in range(zlen // _L):
            zf_v[pl.ds(j * _L, _L)] = zf
        pltpu.sync_copy(zf_v, w_sp.at[pl.ds(s * zlen, zlen)])
        plsc.subcore_barrier()
        pltpu.sync_copy(w_v, w_sp.at[slot_v])
        plsc.subcore_barrier()
        pltpu.sync_copy(w_sp.at[pl.ds(s * zlen, zlen)], zf_v)
        pltpu.sync_copy(zf_v, wp_hbm.at[pl.ds(c * NS + s * zlen, zlen)])

    return route_k(s_pair, w_pair, xc)


def _ffn_body(sp_ref, x_ref, ws0_ref, ws1_ref, w1_ref, w3_ref, w2_ref,
              out_ref):
    b = pl.program_id(0)
    hh = pl.program_id(1)

    @pl.when(b < sp_ref[24])
    def _():
        x = x_ref[...]
        h1 = lax.dot_general(x, w1_ref[0], (((1,), (1,)), ((), ())),
                             preferred_element_type=jnp.float32)
        h3 = lax.dot_general(x, w3_ref[0], (((1,), (1,)), ((), ())),
                             preferred_element_type=jnp.float32)
        h = (h1 / (1.0 + jnp.exp(-h1))) * h3
        y = lax.dot_general(h, w2_ref[0], (((1,), (1,)), ((), ())),
                            preferred_element_type=jnp.float32)
        ys = (ws0_ref[...] + ws1_ref[...]) * y

        @pl.when(hh == 0)
        def _():
            out_ref[...] = ys

        @pl.when(hh != 0)
        def _():
            out_ref[...] += ys


def _ffn(xs, ws0, ws1, w1, w3, w2, sp, NB):
    NS, D = xs.shape
    H = w1.shape[1]
    NH = 2
    HH = H // NH

    def _xmap(b, hh, sp):
        return (jnp.minimum(b, sp[24] - 1), 0)

    grid_spec = pltpu.PrefetchScalarGridSpec(
        num_scalar_prefetch=1,
        grid=(NB, NH),
        in_specs=[
            pl.BlockSpec((_BM, D), _xmap),
            pl.BlockSpec((_BM, 1), _xmap),
            pl.BlockSpec((_BM, 1), _xmap),
            pl.BlockSpec((1, HH, D), lambda b, hh, sp: (sp[b], hh, 0)),
            pl.BlockSpec((1, HH, D), lambda b, hh, sp: (sp[b], hh, 0)),
            pl.BlockSpec((1, D, HH), lambda b, hh, sp: (sp[b], 0, hh)),
        ],
        out_specs=pl.BlockSpec((_BM, D), lambda b, hh, sp: (b, 0)),
    )
    return pl.pallas_call(
        _ffn_body,
        grid_spec=grid_spec,
        out_shape=jax.ShapeDtypeStruct((NS, D), jnp.float32),
        compiler_params=pltpu.CompilerParams(
            dimension_semantics=("arbitrary", "arbitrary"),
        ),
    )(sp, xs, ws0, ws1, w1, w3, w2)


def _sc_combine_call(ys, s0, s1, T):
    NS, D = ys.shape
    tpw = T // _NW
    ch = tpw
    while ch * D * 4 * 2 > 360 * 1024:
        ch //= 2
    nch = tpw // ch
    nvec = D // _L
    mesh = plsc.VectorSubcoreMesh(core_axis_name="c", subcore_axis_name="s")

    @functools.partial(
        pl.kernel, mesh=mesh,
        out_type=jax.ShapeDtypeStruct((T, D), jnp.float32),
        scratch_types=[
            pltpu.VMEM((ch,), jnp.int32),
            pltpu.VMEM((ch,), jnp.int32),
            pltpu.VMEM((ch, D), jnp.float32),
            pltpu.VMEM((ch, D), jnp.float32),
            pltpu.SemaphoreType.DMA,
            pltpu.SemaphoreType.DMA,
        ],
    )
    def combine_k(y_hbm, s0_hbm, s1_hbm, out_hbm, i0_v, i1_v, a_v, b_v,
                  sem0, sem1):
        wid = lax.axis_index("s") * 2 + lax.axis_index("c")
        base = wid * tpw
        for c in range(nch):
            off = base + c * ch
            pltpu.sync_copy(s0_hbm.at[pl.ds(off, ch)], i0_v)
            pltpu.sync_copy(s1_hbm.at[pl.ds(off, ch)], i1_v)
            cp0 = pltpu.async_copy(y_hbm.at[i0_v], a_v, sem0)
            cp1 = pltpu.async_copy(y_hbm.at[i1_v], b_v, sem1)
            cp0.wait()
            cp1.wait()

            def add_row(i, carry):
                for j in range(nvec):
                    sl = pl.ds(j * _L, _L)
                    a_v[i, sl] = a_v[i, sl] + b_v[i, sl]
                return carry

            lax.fori_loop(0, ch, add_row, 0)
            pltpu.sync_copy(a_v, out_hbm.at[pl.ds(off, ch)])

    return combine_k(ys, s0, s1)


def kernel(x, gate_w, w1, w2, w3):
    Bb, S, D = x.shape
    T = Bb * S
    NB = 2 * T // _BM + _E
    NS = NB * _BM
    xf = x.reshape(T, D)
    gwp = jnp.zeros((128, D), jnp.float32).at[:_E].set(gate_w)

    s0, s1, p0, p1, sp2, xc = _gate(xf, gwp)
    s_pair = jnp.concatenate([s0[:, 0], s1[:, 0]])
    w_pair = jnp.concatenate([p0[:, 0], p1[:, 0]])
    sp = sp2[:32, 0]
    xs, wp = _sc_route_call(s_pair, w_pair, xc, NS)
    wp2 = wp.reshape(2, NS)
    ys = _ffn(xs, wp2[0][:, None], wp2[1][:, None], w1, w3, w2, sp, NB)
    out = _sc_combine_call(ys, s0[:, 0], s1[:, 0], T)
    return out.reshape(Bb, S, D)


# speedup vs baseline: 1.3001x; 1.3001x over previous
import functools

import jax
import jax.numpy as jnp
from jax import lax
from jax.experimental import pallas as pl
from jax.experimental.pallas import tpu as pltpu
from jax.experimental.pallas import tpu_sc as plsc

_E = 8
_NEG = -1e30
_BM = 256
_NW = 32
_L = 16
_CPW = 128


def _gate_body(gw_ref, x_ref, s0_ref, s1_ref, p0_ref, p1_ref, sp_ref):
    T = x_ref.shape[0]
    lg = lax.dot_general(x_ref[...], gw_ref[...], (((1,), (1,)), ((), ())),
                         preferred_element_type=jnp.float32)
    lane = lax.broadcasted_iota(jnp.int32, lg.shape, 1)
    lg = jnp.where(lane < _E, lg, _NEG)
    m0 = jnp.max(lg, axis=1, keepdims=True)
    a0 = jnp.min(jnp.where(lg == m0, lane, 128), axis=1, keepdims=True)
    lg1 = jnp.where(lane == a0, _NEG, lg)
    m1 = jnp.max(lg1, axis=1, keepdims=True)
    a1 = jnp.min(jnp.where(lg1 == m1, lane, 128), axis=1, keepdims=True)
    p0 = 1.0 / (1.0 + jnp.exp(m1 - m0))
    p0_ref[...] = p0
    p1_ref[...] = 1.0 - p0

    oh0 = (lane == a0).astype(jnp.bfloat16)
    oh1 = (lane == a1).astype(jnp.bfloat16)
    tri = (lax.broadcasted_iota(jnp.int32, (T, T), 0) >
           lax.broadcasted_iota(jnp.int32, (T, T), 1)
           ).astype(jnp.bfloat16)
    cum0 = lax.dot_general(tri, oh0, (((1,), (0,)), ((), ())),
                           preferred_element_type=jnp.float32)
    cum1 = lax.dot_general(tri, oh1, (((1,), (0,)), ((), ())),
                           preferred_element_type=jnp.float32)
    tot0 = jnp.sum(oh0.astype(jnp.float32), axis=0, keepdims=True)
    tot1 = jnp.sum(oh1.astype(jnp.float32), axis=0, keepdims=True)

    counts = tot0 + tot1
    nbl = jnp.floor((counts + (_BM - 1)) * (1.0 / _BM))
    tril = (lax.broadcasted_iota(jnp.int32, (128, 128), 0) <
            lax.broadcasted_iota(jnp.int32, (128, 128), 1)
            ).astype(jnp.float32)
    off_blk = lax.dot_general(nbl, tril, (((1,), (0,)), ((), ())),
                              preferred_element_type=jnp.float32)
    totblk = jnp.sum(jnp.where(lane[0:1, :] < _E, nbl, 0.0), axis=1,
                     keepdims=True)
    off_rows = off_blk * _BM

    s0 = jnp.sum(jnp.where(lane == a0, off_rows + cum0, 0.0), axis=1,
                 keepdims=True)
    s1 = jnp.sum(jnp.where(lane == a1, off_rows + cum1 + tot0, 0.0), axis=1,
                 keepdims=True)
    s0_ref[...] = s0.astype(jnp.int32)
    s1_ref[...] = s1.astype(jnp.int32)

    bi = lax.broadcasted_iota(jnp.int32, (128, 128), 0).astype(jnp.float32)
    bic = jnp.minimum(bi, totblk - 1.0)
    ec = lax.broadcasted_iota(jnp.int32, (128, 128), 1)
    obc = jnp.broadcast_to(off_blk, (128, 128))
    bem = ((bic >= obc) & (ec >= 1) & (ec < _E)).astype(jnp.float32)
    be = jnp.sum(bem, axis=1, keepdims=True)
    bsub = lax.broadcasted_iota(jnp.int32, (128, 1), 0)
    sp_ref[...] = jnp.where(bsub == 24, totblk, be).astype(jnp.int32)


def _gate(xf, gwp):
    T, D = xf.shape
    return pl.pallas_call(
        _gate_body,
        in_specs=[
            pl.BlockSpec((128, D), lambda: (0, 0)),
            pl.BlockSpec((T, D), lambda: (0, 0)),
        ],
        out_specs=[
            pl.BlockSpec((T, 1), lambda: (0, 0)),
            pl.BlockSpec((T, 1), lambda: (0, 0)),
            pl.BlockSpec((T, 1), lambda: (0, 0)),
            pl.BlockSpec((T, 1), lambda: (0, 0)),
            pl.BlockSpec((128, 1), lambda: (0, 0)),
        ],
        out_shape=[
            jax.ShapeDtypeStruct((T, 1), jnp.int32),
            jax.ShapeDtypeStruct((T, 1), jnp.int32),
            jax.ShapeDtypeStruct((T, 1), jnp.float32),
            jax.ShapeDtypeStruct((T, 1), jnp.float32),
            jax.ShapeDtypeStruct((128, 1), jnp.int32),
        ],
    )(gwp, xf)


def _sc_route_call(s_pair, w_pair, xc, NS):
    P2 = s_pair.shape[0]
    T, D = xc.shape
    zlen = NS // 16
    hch = _CPW // 2
    mesh = plsc.VectorSubcoreMesh(core_axis_name="c", subcore_axis_name="s")

    @functools.partial(
        pl.kernel, mesh=mesh,
        out_type=[
            jax.ShapeDtypeStruct((NS, D), jnp.float32),
            jax.ShapeDtypeStruct((2 * NS,), jnp.float32),
        ],
        scratch_types=[
            pltpu.VMEM((_CPW,), jnp.int32),
            pltpu.VMEM((2, hch), jnp.int32),
            pltpu.VMEM((_CPW,), jnp.float32),
            pltpu.VMEM((hch, D), jnp.float32),
            pltpu.VMEM((hch, D), jnp.float32),
            pltpu.VMEM((NS // 16,), jnp.float32),
            pltpu.VMEM_SHARED((NS,), jnp.float32),
            pltpu.SemaphoreType.DMA,
            pltpu.SemaphoreType.DMA,
        ],
    )
    def route_k(s_hbm, w_hbm, x_hbm, xs_hbm, wp_hbm,
                slot_v, slot2_v, w_v, rowa_v, rowb_v, zf_v, w_sp,
                sa_sem, sb_sem):
        c = lax.axis_index("c")
        s = lax.axis_index("s")
        chunk = c * 16 + s
        base_p = chunk * _CPW
        tok0 = s * _CPW

        pltpu.sync_copy(s_hbm.at[pl.ds(base_p, _CPW)], slot_v)
        pltpu.sync_copy(s_hbm.at[pl.ds(base_p, hch)], slot2_v.at[0])
        pltpu.sync_copy(s_hbm.at[pl.ds(base_p + hch, hch)], slot2_v.at[1])
        pltpu.sync_copy(w_hbm.at[pl.ds(base_p, _CPW)], w_v)

        pltpu.sync_copy(x_hbm.at[pl.ds(tok0, hch)], rowa_v)
        cpa = pltpu.async_copy(rowa_v, xs_hbm.at[slot2_v.at[0]], sa_sem)
        pltpu.sync_copy(x_hbm.at[pl.ds(tok0 + hch, hch)], rowb_v)
        cpb = pltpu.async_copy(rowb_v, xs_hbm.at[slot2_v.at[1]], sb_sem)
        cpa.wait()
        cpb.wait()

        zf = jnp.zeros((_L,), jnp.float32)
        for j in range(zlen // _L):
            zf_v[pl.ds(j * _L, _L)] = zf
        pltpu.sync_copy(zf_v, w_sp.at[pl.ds(s * zlen, zlen)])
        plsc.subcore_barrier()
        pltpu.sync_copy(w_v, w_sp.at[slot_v])
        plsc.subcore_barrier()
        pltpu.sync_copy(w_sp.at[pl.ds(s * zlen, zlen)], zf_v)
        pltpu.sync_copy(zf_v, wp_hbm.at[pl.ds(c * NS + s * zlen, zlen)])

    return route_k(s_pair, w_pair, xc)


def _ffn_body(sp_ref, x_ref, ws0_ref, ws1_ref, w1_ref, w3_ref, w2_ref,
              out_ref):
    b = pl.program_id(0)

    @pl.when(b < sp_ref[24])
    def _():
        x = x_ref[...]
        h1 = lax.dot_general(x, w1_ref[0], (((1,), (1,)), ((), ())),
                             preferred_element_type=jnp.float32)
        h3 = lax.dot_general(x, w3_ref[0], (((1,), (1,)), ((), ())),
                             preferred_element_type=jnp.float32)
        h = (h1 / (1.0 + jnp.exp(-h1))) * h3
        y = lax.dot_general(h, w2_ref[0], (((1,), (1,)), ((), ())),
                            preferred_element_type=jnp.float32)
        out_ref[...] = (ws0_ref[...] + ws1_ref[...]) * y


def _ffn(xs, ws0, ws1, w1, w3, w2, sp, NB):
    NS, D = xs.shape
    H = w1.shape[1]

    def _xmap(b, sp):
        return (jnp.minimum(b, sp[24] - 1), 0)

    grid_spec = pltpu.PrefetchScalarGridSpec(
        num_scalar_prefetch=1,
        grid=(NB,),
        in_specs=[
            pl.BlockSpec((_BM, D), _xmap),
            pl.BlockSpec((_BM, 1), _xmap),
            pl.BlockSpec((_BM, 1), _xmap),
            pl.BlockSpec((1, H, D), lambda b, sp: (sp[b], 0, 0)),
            pl.BlockSpec((1, H, D), lambda b, sp: (sp[b], 0, 0)),
            pl.BlockSpec((1, D, H), lambda b, sp: (sp[b], 0, 0)),
        ],
        out_specs=pl.BlockSpec((_BM, D), lambda b, sp: (b, 0)),
    )
    return pl.pallas_call(
        _ffn_body,
        grid_spec=grid_spec,
        out_shape=jax.ShapeDtypeStruct((NS, D), jnp.float32),
        compiler_params=pltpu.CompilerParams(
            dimension_semantics=("arbitrary",),
        ),
    )(sp, xs, ws0, ws1, w1, w3, w2)


def _sc_combine_call(ys, s0, s1, T):
    NS, D = ys.shape
    tpw = T // _NW
    ch = tpw
    while ch * D * 4 * 2 > 360 * 1024:
        ch //= 2
    nch = tpw // ch
    nvec = D // _L
    mesh = plsc.VectorSubcoreMesh(core_axis_name="c", subcore_axis_name="s")

    @functools.partial(
        pl.kernel, mesh=mesh,
        out_type=jax.ShapeDtypeStruct((T, D), jnp.float32),
        scratch_types=[
            pltpu.VMEM((ch,), jnp.int32),
            pltpu.VMEM((ch,), jnp.int32),
            pltpu.VMEM((ch, D), jnp.float32),
            pltpu.VMEM((ch, D), jnp.float32),
            pltpu.SemaphoreType.DMA,
            pltpu.SemaphoreType.DMA,
        ],
    )
    def combine_k(y_hbm, s0_hbm, s1_hbm, out_hbm, i0_v, i1_v, a_v, b_v,
                  sem0, sem1):
        wid = lax.axis_index("s") * 2 + lax.axis_index("c")
        base = wid * tpw
        for c in range(nch):
            off = base + c * ch
            pltpu.sync_copy(s0_hbm.at[pl.ds(off, ch)], i0_v)
            pltpu.sync_copy(s1_hbm.at[pl.ds(off, ch)], i1_v)
            cp0 = pltpu.async_copy(y_hbm.at[i0_v], a_v, sem0)
            cp1 = pltpu.async_copy(y_hbm.at[i1_v], b_v, sem1)
            cp0.wait()
            cp1.wait()

            def add_row(i, carry):
                for j in range(nvec):
                    sl = pl.ds(j * _L, _L)
                    a_v[i, sl] = a_v[i, sl] + b_v[i, sl]
                return carry

            lax.fori_loop(0, ch, add_row, 0)
            pltpu.sync_copy(a_v, out_hbm.at[pl.ds(off, ch)])

    return combine_k(ys, s0, s1)


def kernel(x, gate_w, w1, w2, w3):
    Bb, S, D = x.shape
    T = Bb * S
    NB = 2 * T // _BM + _E
    NS = NB * _BM
    xf = x.reshape(T, D)
    gwp = jnp.zeros((128, D), jnp.float32).at[:_E].set(gate_w)

    s0, s1, p0, p1, sp2 = _gate(xf, gwp)
    s_pair = jnp.concatenate([s0[:, 0], s1[:, 0]])
    w_pair = jnp.concatenate([p0[:, 0], p1[:, 0]])
    sp = sp2[:32, 0]
    xs, wp = _sc_route_call(s_pair, w_pair, xf, NS)
    wp2 = wp.reshape(2, NS)
    ys = _ffn(xs, wp2[0][:, None], wp2[1][:, None], w1, w3, w2, sp, NB)
    out = _sc_combine_call(ys, s0[:, 0], s1[:, 0], T)
    return out.reshape(Bb, S, D)


# speedup vs baseline: 1.3694x; 1.0533x over previous
import functools

import jax
import jax.numpy as jnp
from jax import lax
from jax.experimental import pallas as pl
from jax.experimental.pallas import tpu as pltpu
from jax.experimental.pallas import tpu_sc as plsc

_E = 8
_NEG = -1e30
_BM = 512
_NW = 32
_L = 16
_CPW = 128


def _gate_body(gw_ref, x_ref, s0_ref, s1_ref, p0_ref, p1_ref, sp_ref):
    T = x_ref.shape[0]
    lg = lax.dot_general(x_ref[...], gw_ref[...], (((1,), (1,)), ((), ())),
                         preferred_element_type=jnp.float32)
    lane = lax.broadcasted_iota(jnp.int32, lg.shape, 1)
    lg = jnp.where(lane < _E, lg, _NEG)
    m0 = jnp.max(lg, axis=1, keepdims=True)
    a0 = jnp.min(jnp.where(lg == m0, lane, 128), axis=1, keepdims=True)
    lg1 = jnp.where(lane == a0, _NEG, lg)
    m1 = jnp.max(lg1, axis=1, keepdims=True)
    a1 = jnp.min(jnp.where(lg1 == m1, lane, 128), axis=1, keepdims=True)
    p0 = 1.0 / (1.0 + jnp.exp(m1 - m0))
    p0_ref[...] = p0
    p1_ref[...] = 1.0 - p0

    oh0 = (lane == a0).astype(jnp.bfloat16)
    oh1 = (lane == a1).astype(jnp.bfloat16)
    tri = (lax.broadcasted_iota(jnp.int32, (T, T), 0) >
           lax.broadcasted_iota(jnp.int32, (T, T), 1)
           ).astype(jnp.bfloat16)
    cum0 = lax.dot_general(tri, oh0, (((1,), (0,)), ((), ())),
                           preferred_element_type=jnp.float32)
    cum1 = lax.dot_general(tri, oh1, (((1,), (0,)), ((), ())),
                           preferred_element_type=jnp.float32)
    tot0 = jnp.sum(oh0.astype(jnp.float32), axis=0, keepdims=True)
    tot1 = jnp.sum(oh1.astype(jnp.float32), axis=0, keepdims=True)

    counts = tot0 + tot1
    nbl = jnp.floor((counts + (_BM - 1)) * (1.0 / _BM))
    tril = (lax.broadcasted_iota(jnp.int32, (128, 128), 0) <
            lax.broadcasted_iota(jnp.int32, (128, 128), 1)
            ).astype(jnp.float32)
    off_blk = lax.dot_general(nbl, tril, (((1,), (0,)), ((), ())),
                              preferred_element_type=jnp.float32)
    totblk = jnp.sum(jnp.where(lane[0:1, :] < _E, nbl, 0.0), axis=1,
                     keepdims=True)
    off_rows = off_blk * _BM

    s0 = jnp.sum(jnp.where(lane == a0, off_rows + cum0, 0.0), axis=1,
                 keepdims=True)
    s1 = jnp.sum(jnp.where(lane == a1, off_rows + cum1 + tot0, 0.0), axis=1,
                 keepdims=True)
    s0_ref[...] = s0.astype(jnp.int32)
    s1_ref[...] = s1.astype(jnp.int32)

    bi = lax.broadcasted_iota(jnp.int32, (128, 128), 0).astype(jnp.float32)
    bic = jnp.minimum(bi, totblk - 1.0)
    ec = lax.broadcasted_iota(jnp.int32, (128, 128), 1)
    obc = jnp.broadcast_to(off_blk, (128, 128))
    bem = ((bic >= obc) & (ec >= 1) & (ec < _E)).astype(jnp.float32)
    be = jnp.sum(bem, axis=1, keepdims=True)
    bsub = lax.broadcasted_iota(jnp.int32, (128, 1), 0)
    sp_ref[...] = jnp.where(bsub == 24, totblk, be).astype(jnp.int32)


def _gate(xf, gwp):
    T, D = xf.shape
    return pl.pallas_call(
        _gate_body,
        in_specs=[
            pl.BlockSpec((128, D), lambda: (0, 0)),
            pl.BlockSpec((T, D), lambda: (0, 0)),
        ],
        out_specs=[
            pl.BlockSpec((T, 1), lambda: (0, 0)),
            pl.BlockSpec((T, 1), lambda: (0, 0)),
            pl.BlockSpec((T, 1), lambda: (0, 0)),
            pl.BlockSpec((T, 1), lambda: (0, 0)),
            pl.BlockSpec((128, 1), lambda: (0, 0)),
        ],
        out_shape=[
            jax.ShapeDtypeStruct((T, 1), jnp.int32),
            jax.ShapeDtypeStruct((T, 1), jnp.int32),
            jax.ShapeDtypeStruct((T, 1), jnp.float32),
            jax.ShapeDtypeStruct((T, 1), jnp.float32),
            jax.ShapeDtypeStruct((128, 1), jnp.int32),
        ],
    )(gwp, xf)


def _sc_route_call(s_pair, w_pair, xc, NS):
    P2 = s_pair.shape[0]
    T, D = xc.shape
    zlen = NS // 16
    hch = _CPW // 2
    mesh = plsc.VectorSubcoreMesh(core_axis_name="c", subcore_axis_name="s")

    @functools.partial(
        pl.kernel, mesh=mesh,
        out_type=[
            jax.ShapeDtypeStruct((NS, D), jnp.float32),
            jax.ShapeDtypeStruct((2 * NS,), jnp.float32),
        ],
        scratch_types=[
            pltpu.VMEM((_CPW,), jnp.int32),
            pltpu.VMEM((2, hch), jnp.int32),
            pltpu.VMEM((_CPW,), jnp.float32),
            pltpu.VMEM((hch, D), jnp.float32),
            pltpu.VMEM((hch, D), jnp.float32),
            pltpu.VMEM((NS // 16,), jnp.float32),
            pltpu.VMEM_SHARED((NS,), jnp.float32),
            pltpu.SemaphoreType.DMA,
            pltpu.SemaphoreType.DMA,
        ],
    )
    def route_k(s_hbm, w_hbm, x_hbm, xs_hbm, wp_hbm,
                slot_v, slot2_v, w_v, rowa_v, rowb_v, zf_v, w_sp,
                sa_sem, sb_sem):
        c = lax.axis_index("c")
        s = lax.axis_index("s")
        chunk = c * 16 + s
        base_p = chunk * _CPW
        tok0 = s * _CPW

        pltpu.sync_copy(s_hbm.at[pl.ds(base_p, _CPW)], slot_v)
        pltpu.sync_copy(s_hbm.at[pl.ds(base_p, hch)], slot2_v.at[0])
        pltpu.sync_copy(s_hbm.at[pl.ds(base_p + hch, hch)], slot2_v.at[1])
        pltpu.sync_copy(w_hbm.at[pl.ds(base_p, _CPW)], w_v)

        pltpu.sync_copy(x_hbm.at[pl.ds(tok0, hch)], rowa_v)
        cpa = pltpu.async_copy(rowa_v, xs_hbm.at[slot2_v.at[0]], sa_sem)
        pltpu.sync_copy(x_hbm.at[pl.ds(tok0 + hch, hch)], rowb_v)
        cpb = pltpu.async_copy(rowb_v, xs_hbm.at[slot2_v.at[1]], sb_sem)
        cpa.wait()
        cpb.wait()

        zf = jnp.zeros((_L,), jnp.float32)
        for j in range(zlen // _L):
            zf_v[pl.ds(j * _L, _L)] = zf
        pltpu.sync_copy(zf_v, w_sp.at[pl.ds(s * zlen, zlen)])
        plsc.subcore_barrier()
        pltpu.sync_copy(w_v, w_sp.at[slot_v])
        plsc.subcore_barrier()
        pltpu.sync_copy(w_sp.at[pl.ds(s * zlen, zlen)], zf_v)
        pltpu.sync_copy(zf_v, wp_hbm.at[pl.ds(c * NS + s * zlen, zlen)])

    return route_k(s_pair, w_pair, xc)


def _ffn_body(sp_ref, x_ref, ws0_ref, ws1_ref, w1_ref, w3_ref, w2_ref,
              out_ref):
    b = pl.program_id(0)

    @pl.when(b < sp_ref[24])
    def _():
        x = x_ref[...]
        h1 = lax.dot_general(x, w1_ref[0], (((1,), (1,)), ((), ())),
                             preferred_element_type=jnp.float32)
        h3 = lax.dot_general(x, w3_ref[0], (((1,), (1,)), ((), ())),
                             preferred_element_type=jnp.float32)
        h = (h1 / (1.0 + jnp.exp(-h1))) * h3
        y = lax.dot_general(h, w2_ref[0], (((1,), (1,)), ((), ())),
                            preferred_element_type=jnp.float32)
        out_ref[...] = (ws0_ref[...] + ws1_ref[...]) * y


def _ffn(xs, ws0, ws1, w1, w3, w2, sp, NB):
    NS, D = xs.shape
    H = w1.shape[1]

    def _xmap(b, sp):
        return (jnp.minimum(b, sp[24] - 1), 0)

    grid_spec = pltpu.PrefetchScalarGridSpec(
        num_scalar_prefetch=1,
        grid=(NB,),
        in_specs=[
            pl.BlockSpec((_BM, D), _xmap),
            pl.BlockSpec((_BM, 1), _xmap),
            pl.BlockSpec((_BM, 1), _xmap),
            pl.BlockSpec((1, H, D), lambda b, sp: (sp[b], 0, 0)),
            pl.BlockSpec((1, H, D), lambda b, sp: (sp[b], 0, 0)),
            pl.BlockSpec((1, D, H), lambda b, sp: (sp[b], 0, 0)),
        ],
        out_specs=pl.BlockSpec((_BM, D), lambda b, sp: (b, 0)),
    )
    return pl.pallas_call(
        _ffn_body,
        grid_spec=grid_spec,
        out_shape=jax.ShapeDtypeStruct((NS, D), jnp.float32),
        compiler_params=pltpu.CompilerParams(
            dimension_semantics=("arbitrary",),
        ),
    )(sp, xs, ws0, ws1, w1, w3, w2)


def _sc_combine_call(ys, s0, s1, T):
    NS, D = ys.shape
    tpw = T // _NW
    ch = tpw
    while ch * D * 4 * 2 > 360 * 1024:
        ch //= 2
    nch = tpw // ch
    nvec = D // _L
    mesh = plsc.VectorSubcoreMesh(core_axis_name="c", subcore_axis_name="s")

    @functools.partial(
        pl.kernel, mesh=mesh,
        out_type=jax.ShapeDtypeStruct((T, D), jnp.float32),
        scratch_types=[
            pltpu.VMEM((ch,), jnp.int32),
            pltpu.VMEM((ch,), jnp.int32),
            pltpu.VMEM((ch, D), jnp.float32),
            pltpu.VMEM((ch, D), jnp.float32),
            pltpu.SemaphoreType.DMA,
            pltpu.SemaphoreType.DMA,
        ],
    )
    def combine_k(y_hbm, s0_hbm, s1_hbm, out_hbm, i0_v, i1_v, a_v, b_v,
                  sem0, sem1):
        wid = lax.axis_index("s") * 2 + lax.axis_index("c")
        base = wid * tpw
        for c in range(nch):
            off = base + c * ch
            pltpu.sync_copy(s0_hbm.at[pl.ds(off, ch)], i0_v)
            pltpu.sync_copy(s1_hbm.at[pl.ds(off, ch)], i1_v)
            cp0 = pltpu.async_copy(y_hbm.at[i0_v], a_v, sem0)
            cp1 = pltpu.async_copy(y_hbm.at[i1_v], b_v, sem1)
            cp0.wait()
            cp1.wait()

            def add_row(i, carry):
                for j in range(nvec):
                    sl = pl.ds(j * _L, _L)
                    a_v[i, sl] = a_v[i, sl] + b_v[i, sl]
                return carry

            lax.fori_loop(0, ch, add_row, 0)
            pltpu.sync_copy(a_v, out_hbm.at[pl.ds(off, ch)])

    return combine_k(ys, s0, s1)


def kernel(x, gate_w, w1, w2, w3):
    Bb, S, D = x.shape
    T = Bb * S
    NB = 2 * T // _BM + _E
    NS = NB * _BM
    xf = x.reshape(T, D)
    gwp = jnp.zeros((128, D), jnp.float32).at[:_E].set(gate_w)

    s0, s1, p0, p1, sp2 = _gate(xf, gwp)
    s_pair = jnp.concatenate([s0[:, 0], s1[:, 0]])
    w_pair = jnp.concatenate([p0[:, 0], p1[:, 0]])
    sp = sp2[:32, 0]
    xs, wp = _sc_route_call(s_pair, w_pair, xf, NS)
    wp2 = wp.reshape(2, NS)
    ys = _ffn(xs, wp2[0][:, None], wp2[1][:, None], w1, w3, w2, sp, NB)
    out = _sc_combine_call(ys, s0[:, 0], s1[:, 0], T)
    return out.reshape(Bb, S, D)
